# baseline (speedup 1.0000x reference)
import functools

import jax
import jax.numpy as jnp
from jax import lax
from jax.experimental import pallas as pl
from jax.experimental.pallas import tpu as pltpu
from jax.experimental.pallas import tpu_sc as plsc

B, S, D = 16, 4096, 512
SB = 512
S_BLKS = S // SB
EPS = 1e-5

NC, NS, L = 2, 16, 16
NW = NC * NS
CHUNK = 64
NSL = D // L
KG = 4


def _sc_stats_body(x_hbm, lens_hbm, sum_hbm, sq_hbm, lens_v, buf, acc_s,
                   acc_q, sem):
    wid = lax.axis_index("s") * NC + lax.axis_index("c")
    pltpu.sync_copy(lens_hbm, lens_v)
    lv = lens_v[...]
    total = jnp.int32(0)
    for b in range(B):
        total = total + lv[b]
    tstart = wid * total // NW
    tend = (wid + 1) * total // NW

    for k in range(NSL):
        acc_s[pl.ds(k * L, L)] = jnp.zeros((L,), jnp.float32)
        acc_q[pl.ds(k * L, L)] = jnp.zeros((L,), jnp.float32)

    def row_body(b, pre_b):
        len_b = jnp.int32(0)
        for i in range(B):
            len_b = len_b + jnp.where(b == i, lv[i], 0)
        lo = jnp.maximum(tstart - pre_b, 0)
        hi = jnp.minimum(tend - pre_b, len_b)

        s00 = jnp.minimum((lo // 8) * 8, S - CHUNK)
        nch = jnp.where(hi > lo, (hi - s00 + CHUNK - 1) // CHUNK, 0)

        def s0_of(i):
            return pl.multiple_of(jnp.minimum(s00 + i * CHUNK, S - CHUNK), 8)

        def start_dma(i):
            slot = lax.rem(i, 2)
            pltpu.make_async_copy(x_hbm.at[b, pl.ds(s0_of(i), CHUNK)],
                                  buf.at[slot], sem.at[slot]).start()

        @pl.when(nch > 0)
        def _prime():
            start_dma(0)

        def chunk_body(ci, _):
            slot = lax.rem(ci, 2)
            pltpu.make_async_copy(x_hbm.at[b, pl.ds(s0_of(ci), CHUNK)],
                                  buf.at[slot], sem.at[slot]).wait()

            @pl.when(ci + 1 < nch)
            def _prefetch():
                start_dma(ci + 1)

            s0 = s0_of(ci)
            p = jnp.where(ci == 0, lo, s00 + ci * CHUNK)
            off = p - s0
            lim = jnp.minimum(hi - s0, CHUNK)

            def zero_tok(t, _):
                for k in range(NSL):
                    buf[slot, t, pl.ds(k * L, L)] = jnp.zeros((L,),
                                                              jnp.float32)
                return 0

            lax.fori_loop(0, off, zero_tok, 0)
            lax.fori_loop(lim, CHUNK, zero_tok, 0)

            for g in range(NSL // KG):
                sls = [pl.ds((g * KG + u) * L, L) for u in range(KG)]

                def tok(t, carry, sls=sls, slot=slot):
                    vs = [buf[slot, t, sl] for sl in sls]
                    ss = [carry[u] + vs[u] for u in range(KG)]
                    qs = [carry[KG + u] + vs[u] * vs[u] for u in range(KG)]
                    return tuple(ss + qs)

                init = tuple([acc_s[sl] for sl in sls]
                             + [acc_q[sl] for sl in sls])
                res = lax.fori_loop(0, CHUNK, tok, init, unroll=2)
                for u in range(KG):
                    acc_s[sls[u]] = res[u]
                    acc_q[sls[u]] = res[KG + u]
            return 0

        lax.fori_loop(0, nch, chunk_body, 0)
        return pre_b + len_b

    lax.fori_loop(0, B, row_body, jnp.int32(0))
    wbase = pl.multiple_of(wid * D, 8)
    pltpu.sync_copy(acc_s, sum_hbm.at[pl.ds(wbase, D)])
    pltpu.sync_copy(acc_q, sq_hbm.at[pl.ds(wbase, D)])


_sc_stats = functools.partial(
    pl.kernel,
    out_type=[
        jax.ShapeDtypeStruct((NW * D,), jnp.float32),
        jax.ShapeDtypeStruct((NW * D,), jnp.float32),
    ],
    mesh=plsc.VectorSubcoreMesh(core_axis_name="c", subcore_axis_name="s",
                                num_cores=NC, num_subcores=NS),
    scratch_types=[
        pltpu.VMEM((L,), jnp.int32),
        pltpu.VMEM((2, CHUNK, D), jnp.float32),
        pltpu.VMEM((D,), jnp.float32),
        pltpu.VMEM((D,), jnp.float32),
        pltpu.SemaphoreType.DMA((2,)),
    ],
)(_sc_stats_body)


def _finalize_body(lens_ref, sum_ref, sq_ref, w_ref, b_ref,
                   scale_ref, shift_ref):
    cnt_i = lax.fori_loop(0, B, lambda i, a: a + lens_ref[i], 0)
    cnt = jnp.maximum(cnt_i, 1).astype(jnp.float32)
    mean = jnp.sum(sum_ref[...], axis=0) / cnt
    var = jnp.maximum(jnp.sum(sq_ref[...], axis=0) / cnt - mean * mean, 0.0)
    scale_ref[0, :] = w_ref[0, :] * lax.rsqrt(var + EPS)
    shift_ref[0, :] = b_ref[0, :] - mean * scale_ref[0, :]


def _norm_body(lens_ref, x_ref, scale_ref, shift_ref, o_ref):
    b = pl.program_id(0)
    j = pl.program_id(1)
    rel = lens_ref[b] - j * SB
    x = x_ref[0]
    iota = lax.broadcasted_iota(jnp.int32, (SB, 1), 0)
    valid = iota < rel
    o_ref[0] = jnp.where(valid,
                         x * scale_ref[0, :][None, :]
                         + shift_ref[0, :][None, :], 0.0)


@jax.jit
def _vlbn(x, lens32, weight, bias):
    sums, sqs = _sc_stats(x, lens32)
    sums = sums.reshape(NW, D)
    sqs = sqs.reshape(NW, D)

    lens_spec = pl.BlockSpec(memory_space=pltpu.SMEM)
    scale, shift = pl.pallas_call(
        _finalize_body,
        in_specs=[lens_spec] + [pl.BlockSpec((NW, D), lambda: (0, 0))] * 2
                 + [pl.BlockSpec((1, D), lambda: (0, 0))] * 2,
        out_specs=[pl.BlockSpec((1, D), lambda: (0, 0))] * 2,
        out_shape=[jax.ShapeDtypeStruct((1, D), jnp.float32)] * 2,
    )(lens32, sums, sqs, weight.reshape(1, D), bias.reshape(1, D))

    def x_map(b, j, lens):
        return (b, jnp.where(j * SB < lens[b], j, 0), 0)

    grid_spec = pltpu.PrefetchScalarGridSpec(
        num_scalar_prefetch=1,
        grid=(B, S_BLKS),
        in_specs=[
            pl.BlockSpec((1, SB, D), x_map),
            pl.BlockSpec((1, D), lambda b, j, lens: (0, 0)),
            pl.BlockSpec((1, D), lambda b, j, lens: (0, 0)),
        ],
        out_specs=pl.BlockSpec((1, SB, D), lambda b, j, lens: (b, j, 0)),
    )
    out = pl.pallas_call(
        _norm_body,
        grid_spec=grid_spec,
        out_shape=jax.ShapeDtypeStruct((B, S, D), jnp.float32),
    )(lens32, x, scale, shift)
    return out


def kernel(inputs, seq_lens, weight, bias):
    lens32 = seq_lens.astype(jnp.int32)
    with jax.enable_x64(False):
        return _vlbn(inputs.astype(jnp.float32), lens32,
                     weight.astype(jnp.float32), bias.astype(jnp.float32))


# speedup vs baseline: 1.1762x; 1.1762x over previous
import functools

import jax
import jax.numpy as jnp
from jax import lax
from jax.experimental import pallas as pl
from jax.experimental.pallas import tpu as pltpu
from jax.experimental.pallas import tpu_sc as plsc

B, S, D = 16, 4096, 512
SB = 1024
S_BLKS = S // SB
EPS = 1e-5

NC, NS, L = 2, 16, 16
NW = NC * NS
CHUNK = 64
NSL = D // L
KG = 4


def _sc_stats_body(x_hbm, lens_hbm, sum_hbm, sq_hbm, lens_v, buf, acc_s,
                   acc_q, sem):
    wid = lax.axis_index("s") * NC + lax.axis_index("c")
    pltpu.sync_copy(lens_hbm, lens_v)
    lv = lens_v[...]
    total = jnp.int32(0)
    for b in range(B):
        total = total + lv[b]
    tstart = wid * total // NW
    tend = (wid + 1) * total // NW

    for k in range(NSL):
        acc_s[pl.ds(k * L, L)] = jnp.zeros((L,), jnp.float32)
        acc_q[pl.ds(k * L, L)] = jnp.zeros((L,), jnp.float32)

    def row_body(b, pre_b):
        len_b = jnp.int32(0)
        for i in range(B):
            len_b = len_b + jnp.where(b == i, lv[i], 0)
        lo = jnp.maximum(tstart - pre_b, 0)
        hi = jnp.minimum(tend - pre_b, len_b)

        s00 = jnp.minimum((lo // 8) * 8, S - CHUNK)
        nch = jnp.where(hi > lo, (hi - s00 + CHUNK - 1) // CHUNK, 0)

        def s0_of(i):
            return pl.multiple_of(jnp.minimum(s00 + i * CHUNK, S - CHUNK), 8)

        def start_dma(i):
            slot = lax.rem(i, 2)
            pltpu.make_async_copy(x_hbm.at[b, pl.ds(s0_of(i), CHUNK)],
                                  buf.at[slot], sem.at[slot]).start()

        @pl.when(nch > 0)
        def _prime():
            start_dma(0)

        def chunk_body(ci, _):
            slot = lax.rem(ci, 2)
            pltpu.make_async_copy(x_hbm.at[b, pl.ds(s0_of(ci), CHUNK)],
                                  buf.at[slot], sem.at[slot]).wait()

            @pl.when(ci + 1 < nch)
            def _prefetch():
                start_dma(ci + 1)

            s0 = s0_of(ci)
            p = jnp.where(ci == 0, lo, s00 + ci * CHUNK)
            off = p - s0
            lim = jnp.minimum(hi - s0, CHUNK)

            def zero_tok(t, _):
                for k in range(NSL):
                    buf[slot, t, pl.ds(k * L, L)] = jnp.zeros((L,),
                                                              jnp.float32)
                return 0

            lax.fori_loop(0, off, zero_tok, 0)
            lax.fori_loop(lim, CHUNK, zero_tok, 0)

            for g in range(NSL // KG):
                sls = [pl.ds((g * KG + u) * L, L) for u in range(KG)]

                def tok(t, carry, sls=sls, slot=slot):
                    vs = [buf[slot, t, sl] for sl in sls]
                    ss = [carry[u] + vs[u] for u in range(KG)]
                    qs = [carry[KG + u] + vs[u] * vs[u] for u in range(KG)]
                    return tuple(ss + qs)

                init = tuple([acc_s[sl] for sl in sls]
                             + [acc_q[sl] for sl in sls])
                res = lax.fori_loop(0, CHUNK, tok, init, unroll=2)
                for u in range(KG):
                    acc_s[sls[u]] = res[u]
                    acc_q[sls[u]] = res[KG + u]
            return 0

        lax.fori_loop(0, nch, chunk_body, 0)
        return pre_b + len_b

    lax.fori_loop(0, B, row_body, jnp.int32(0))
    wbase = pl.multiple_of(wid * D, 8)
    pltpu.sync_copy(acc_s, sum_hbm.at[pl.ds(wbase, D)])
    pltpu.sync_copy(acc_q, sq_hbm.at[pl.ds(wbase, D)])


_sc_stats = functools.partial(
    pl.kernel,
    out_type=[
        jax.ShapeDtypeStruct((NW * D,), jnp.float32),
        jax.ShapeDtypeStruct((NW * D,), jnp.float32),
    ],
    mesh=plsc.VectorSubcoreMesh(core_axis_name="c", subcore_axis_name="s",
                                num_cores=NC, num_subcores=NS),
    scratch_types=[
        pltpu.VMEM((L,), jnp.int32),
        pltpu.VMEM((2, CHUNK, D), jnp.float32),
        pltpu.VMEM((D,), jnp.float32),
        pltpu.VMEM((D,), jnp.float32),
        pltpu.SemaphoreType.DMA((2,)),
    ],
)(_sc_stats_body)


def _finalize_body(lens_ref, sum_ref, sq_ref, w_ref, b_ref,
                   scale_ref, shift_ref):
    cnt_i = lax.fori_loop(0, B, lambda i, a: a + lens_ref[i], 0)
    cnt = jnp.maximum(cnt_i, 1).astype(jnp.float32)
    mean = jnp.sum(sum_ref[...], axis=0) / cnt
    var = jnp.maximum(jnp.sum(sq_ref[...], axis=0) / cnt - mean * mean, 0.0)
    scale_ref[0, :] = w_ref[0, :] * lax.rsqrt(var + EPS)
    shift_ref[0, :] = b_ref[0, :] - mean * scale_ref[0, :]


def _norm_body(lens_ref, x_ref, scale_ref, shift_ref, o_ref):
    b = pl.program_id(0)
    j = pl.program_id(1)
    rel = lens_ref[b] - j * SB
    x = x_ref[0]
    iota = lax.broadcasted_iota(jnp.int32, (SB, 1), 0)
    valid = iota < rel
    o_ref[0] = jnp.where(valid,
                         x * scale_ref[0, :][None, :]
                         + shift_ref[0, :][None, :], 0.0)


@jax.jit
def _vlbn(x, lens32, weight, bias):
    sums, sqs = _sc_stats(x, lens32)
    sums = sums.reshape(NW, D)
    sqs = sqs.reshape(NW, D)

    lens_spec = pl.BlockSpec(memory_space=pltpu.SMEM)
    scale, shift = pl.pallas_call(
        _finalize_body,
        in_specs=[lens_spec] + [pl.BlockSpec((NW, D), lambda: (0, 0))] * 2
                 + [pl.BlockSpec((1, D), lambda: (0, 0))] * 2,
        out_specs=[pl.BlockSpec((1, D), lambda: (0, 0))] * 2,
        out_shape=[jax.ShapeDtypeStruct((1, D), jnp.float32)] * 2,
    )(lens32, sums, sqs, weight.reshape(1, D), bias.reshape(1, D))

    def x_map(b, j, lens):
        return (b, jnp.where(j * SB < lens[b], j, 0), 0)

    grid_spec = pltpu.PrefetchScalarGridSpec(
        num_scalar_prefetch=1,
        grid=(B, S_BLKS),
        in_specs=[
            pl.BlockSpec((1, SB, D), x_map),
            pl.BlockSpec((1, D), lambda b, j, lens: (0, 0)),
            pl.BlockSpec((1, D), lambda b, j, lens: (0, 0)),
        ],
        out_specs=pl.BlockSpec((1, SB, D), lambda b, j, lens: (b, j, 0)),
    )
    out = pl.pallas_call(
        _norm_body,
        grid_spec=grid_spec,
        out_shape=jax.ShapeDtypeStruct((B, S, D), jnp.float32),
    )(lens32, x, scale, shift)
    return out


def kernel(inputs, seq_lens, weight, bias):
    lens32 = seq_lens.astype(jnp.int32)
    with jax.enable_x64(False):
        return _vlbn(inputs.astype(jnp.float32), lens32,
                     weight.astype(jnp.float32), bias.astype(jnp.float32))


# speedup vs baseline: 1.2410x; 1.0550x over previous
import functools

import jax
import jax.numpy as jnp
from jax import lax
from jax.experimental import pallas as pl
from jax.experimental.pallas import tpu as pltpu
from jax.experimental.pallas import tpu_sc as plsc

B, S, D = 16, 4096, 512
SB = 2048
S_BLKS = S // SB
EPS = 1e-5

NC, NS, L = 2, 16, 16
NW = NC * NS
CHUNK = 64
NSL = D // L
KG = 4


def _sc_stats_body(x_hbm, lens_hbm, sum_hbm, sq_hbm, lens_v, buf, acc_s,
                   acc_q, sem):
    wid = lax.axis_index("s") * NC + lax.axis_index("c")
    pltpu.sync_copy(lens_hbm, lens_v)
    lv = lens_v[...]
    total = jnp.int32(0)
    for b in range(B):
        total = total + lv[b]
    tstart = wid * total // NW
    tend = (wid + 1) * total // NW

    for k in range(NSL):
        acc_s[pl.ds(k * L, L)] = jnp.zeros((L,), jnp.float32)
        acc_q[pl.ds(k * L, L)] = jnp.zeros((L,), jnp.float32)

    def row_body(b, pre_b):
        len_b = jnp.int32(0)
        for i in range(B):
            len_b = len_b + jnp.where(b == i, lv[i], 0)
        lo = jnp.maximum(tstart - pre_b, 0)
        hi = jnp.minimum(tend - pre_b, len_b)

        s00 = jnp.minimum((lo // 8) * 8, S - CHUNK)
        nch = jnp.where(hi > lo, (hi - s00 + CHUNK - 1) // CHUNK, 0)

        def s0_of(i):
            return pl.multiple_of(jnp.minimum(s00 + i * CHUNK, S - CHUNK), 8)

        def start_dma(i):
            slot = lax.rem(i, 2)
            pltpu.make_async_copy(x_hbm.at[b, pl.ds(s0_of(i), CHUNK)],
                                  buf.at[slot], sem.at[slot]).start()

        @pl.when(nch > 0)
        def _prime():
            start_dma(0)

        def chunk_body(ci, _):
            slot = lax.rem(ci, 2)
            pltpu.make_async_copy(x_hbm.at[b, pl.ds(s0_of(ci), CHUNK)],
                                  buf.at[slot], sem.at[slot]).wait()

            @pl.when(ci + 1 < nch)
            def _prefetch():
                start_dma(ci + 1)

            s0 = s0_of(ci)
            p = jnp.where(ci == 0, lo, s00 + ci * CHUNK)
            off = p - s0
            lim = jnp.minimum(hi - s0, CHUNK)

            def zero_tok(t, _):
                for k in range(NSL):
                    buf[slot, t, pl.ds(k * L, L)] = jnp.zeros((L,),
                                                              jnp.float32)
                return 0

            lax.fori_loop(0, off, zero_tok, 0)
            lax.fori_loop(lim, CHUNK, zero_tok, 0)

            for g in range(NSL // KG):
                sls = [pl.ds((g * KG + u) * L, L) for u in range(KG)]

                def tok(t, carry, sls=sls, slot=slot):
                    vs = [buf[slot, t, sl] for sl in sls]
                    ss = [carry[u] + vs[u] for u in range(KG)]
                    qs = [carry[KG + u] + vs[u] * vs[u] for u in range(KG)]
                    return tuple(ss + qs)

                init = tuple([acc_s[sl] for sl in sls]
                             + [acc_q[sl] for sl in sls])
                res = lax.fori_loop(0, CHUNK, tok, init, unroll=2)
                for u in range(KG):
                    acc_s[sls[u]] = res[u]
                    acc_q[sls[u]] = res[KG + u]
            return 0

        lax.fori_loop(0, nch, chunk_body, 0)
        return pre_b + len_b

    lax.fori_loop(0, B, row_body, jnp.int32(0))
    wbase = pl.multiple_of(wid * D, 8)
    pltpu.sync_copy(acc_s, sum_hbm.at[pl.ds(wbase, D)])
    pltpu.sync_copy(acc_q, sq_hbm.at[pl.ds(wbase, D)])


_sc_stats = functools.partial(
    pl.kernel,
    out_type=[
        jax.ShapeDtypeStruct((NW * D,), jnp.float32),
        jax.ShapeDtypeStruct((NW * D,), jnp.float32),
    ],
    mesh=plsc.VectorSubcoreMesh(core_axis_name="c", subcore_axis_name="s",
                                num_cores=NC, num_subcores=NS),
    scratch_types=[
        pltpu.VMEM((L,), jnp.int32),
        pltpu.VMEM((2, CHUNK, D), jnp.float32),
        pltpu.VMEM((D,), jnp.float32),
        pltpu.VMEM((D,), jnp.float32),
        pltpu.SemaphoreType.DMA((2,)),
    ],
)(_sc_stats_body)


def _finalize_body(lens_ref, sum_ref, sq_ref, w_ref, b_ref,
                   scale_ref, shift_ref):
    cnt_i = lax.fori_loop(0, B, lambda i, a: a + lens_ref[i], 0)
    cnt = jnp.maximum(cnt_i, 1).astype(jnp.float32)
    mean = jnp.sum(sum_ref[...], axis=0) / cnt
    var = jnp.maximum(jnp.sum(sq_ref[...], axis=0) / cnt - mean * mean, 0.0)
    scale_ref[0, :] = w_ref[0, :] * lax.rsqrt(var + EPS)
    shift_ref[0, :] = b_ref[0, :] - mean * scale_ref[0, :]


def _norm_body(lens_ref, x_ref, scale_ref, shift_ref, o_ref):
    b = pl.program_id(0)
    j = pl.program_id(1)
    rel = lens_ref[b] - j * SB
    x = x_ref[0]
    iota = lax.broadcasted_iota(jnp.int32, (SB, 1), 0)
    valid = iota < rel
    o_ref[0] = jnp.where(valid,
                         x * scale_ref[0, :][None, :]
                         + shift_ref[0, :][None, :], 0.0)


@jax.jit
def _vlbn(x, lens32, weight, bias):
    sums, sqs = _sc_stats(x, lens32)
    sums = sums.reshape(NW, D)
    sqs = sqs.reshape(NW, D)

    lens_spec = pl.BlockSpec(memory_space=pltpu.SMEM)
    scale, shift = pl.pallas_call(
        _finalize_body,
        in_specs=[lens_spec] + [pl.BlockSpec((NW, D), lambda: (0, 0))] * 2
                 + [pl.BlockSpec((1, D), lambda: (0, 0))] * 2,
        out_specs=[pl.BlockSpec((1, D), lambda: (0, 0))] * 2,
        out_shape=[jax.ShapeDtypeStruct((1, D), jnp.float32)] * 2,
    )(lens32, sums, sqs, weight.reshape(1, D), bias.reshape(1, D))

    def x_map(b, j, lens):
        lastv = jnp.maximum((lens[b] + SB - 1) // SB - 1, 0)
        return (b, jnp.minimum(j, lastv), 0)

    grid_spec = pltpu.PrefetchScalarGridSpec(
        num_scalar_prefetch=1,
        grid=(B, S_BLKS),
        in_specs=[
            pl.BlockSpec((1, SB, D), x_map),
            pl.BlockSpec((1, D), lambda b, j, lens: (0, 0)),
            pl.BlockSpec((1, D), lambda b, j, lens: (0, 0)),
        ],
        out_specs=pl.BlockSpec((1, SB, D), lambda b, j, lens: (b, j, 0)),
    )
    out = pl.pallas_call(
        _norm_body,
        grid_spec=grid_spec,
        out_shape=jax.ShapeDtypeStruct((B, S, D), jnp.float32),
    )(lens32, x, scale, shift)
    return out


def kernel(inputs, seq_lens, weight, bias):
    lens32 = seq_lens.astype(jnp.int32)
    with jax.enable_x64(False):
        return _vlbn(inputs.astype(jnp.float32), lens32,
                     weight.astype(jnp.float32), bias.astype(jnp.float32))


# speedup vs baseline: 1.2504x; 1.0076x over previous
import functools

import jax
import jax.numpy as jnp
from jax import lax
from jax.experimental import pallas as pl
from jax.experimental.pallas import tpu as pltpu
from jax.experimental.pallas import tpu_sc as plsc

B, S, D = 16, 4096, 512
SB = 4096
S_BLKS = S // SB
EPS = 1e-5

NC, NS, L = 2, 16, 16
NW = NC * NS
CHUNK = 64
NSL = D // L
KG = 4


def _sc_stats_body(x_hbm, lens_hbm, sum_hbm, sq_hbm, lens_v, buf, acc_s,
                   acc_q, sem):
    wid = lax.axis_index("s") * NC + lax.axis_index("c")
    pltpu.sync_copy(lens_hbm, lens_v)
    lv = lens_v[...]
    total = jnp.int32(0)
    for b in range(B):
        total = total + lv[b]
    tstart = wid * total // NW
    tend = (wid + 1) * total // NW

    for k in range(NSL):
        acc_s[pl.ds(k * L, L)] = jnp.zeros((L,), jnp.float32)
        acc_q[pl.ds(k * L, L)] = jnp.zeros((L,), jnp.float32)

    def row_body(b, pre_b):
        len_b = jnp.int32(0)
        for i in range(B):
            len_b = len_b + jnp.where(b == i, lv[i], 0)
        lo = jnp.maximum(tstart - pre_b, 0)
        hi = jnp.minimum(tend - pre_b, len_b)

        s00 = jnp.minimum((lo // 8) * 8, S - CHUNK)
        nch = jnp.where(hi > lo, (hi - s00 + CHUNK - 1) // CHUNK, 0)

        def s0_of(i):
            return pl.multiple_of(jnp.minimum(s00 + i * CHUNK, S - CHUNK), 8)

        def start_dma(i):
            slot = lax.rem(i, 2)
            pltpu.make_async_copy(x_hbm.at[b, pl.ds(s0_of(i), CHUNK)],
                                  buf.at[slot], sem.at[slot]).start()

        @pl.when(nch > 0)
        def _prime():
            start_dma(0)

        def chunk_body(ci, _):
            slot = lax.rem(ci, 2)
            pltpu.make_async_copy(x_hbm.at[b, pl.ds(s0_of(ci), CHUNK)],
                                  buf.at[slot], sem.at[slot]).wait()

            @pl.when(ci + 1 < nch)
            def _prefetch():
                start_dma(ci + 1)

            s0 = s0_of(ci)
            p = jnp.where(ci == 0, lo, s00 + ci * CHUNK)
            off = p - s0
            lim = jnp.minimum(hi - s0, CHUNK)

            def zero_tok(t, _):
                for k in range(NSL):
                    buf[slot, t, pl.ds(k * L, L)] = jnp.zeros((L,),
                                                              jnp.float32)
                return 0

            lax.fori_loop(0, off, zero_tok, 0)
            lax.fori_loop(lim, CHUNK, zero_tok, 0)

            for g in range(NSL // KG):
                sls = [pl.ds((g * KG + u) * L, L) for u in range(KG)]

                def tok(t, carry, sls=sls, slot=slot):
                    vs = [buf[slot, t, sl] for sl in sls]
                    ss = [carry[u] + vs[u] for u in range(KG)]
                    qs = [carry[KG + u] + vs[u] * vs[u] for u in range(KG)]
                    return tuple(ss + qs)

                init = tuple([acc_s[sl] for sl in sls]
                             + [acc_q[sl] for sl in sls])
                res = lax.fori_loop(0, CHUNK, tok, init, unroll=2)
                for u in range(KG):
                    acc_s[sls[u]] = res[u]
                    acc_q[sls[u]] = res[KG + u]
            return 0

        lax.fori_loop(0, nch, chunk_body, 0)
        return pre_b + len_b

    lax.fori_loop(0, B, row_body, jnp.int32(0))
    wbase = pl.multiple_of(wid * D, 8)
    pltpu.sync_copy(acc_s, sum_hbm.at[pl.ds(wbase, D)])
    pltpu.sync_copy(acc_q, sq_hbm.at[pl.ds(wbase, D)])


_sc_stats = functools.partial(
    pl.kernel,
    out_type=[
        jax.ShapeDtypeStruct((NW * D,), jnp.float32),
        jax.ShapeDtypeStruct((NW * D,), jnp.float32),
    ],
    mesh=plsc.VectorSubcoreMesh(core_axis_name="c", subcore_axis_name="s",
                                num_cores=NC, num_subcores=NS),
    scratch_types=[
        pltpu.VMEM((L,), jnp.int32),
        pltpu.VMEM((2, CHUNK, D), jnp.float32),
        pltpu.VMEM((D,), jnp.float32),
        pltpu.VMEM((D,), jnp.float32),
        pltpu.SemaphoreType.DMA((2,)),
    ],
)(_sc_stats_body)


def _finalize_body(lens_ref, sum_ref, sq_ref, w_ref, b_ref,
                   scale_ref, shift_ref):
    cnt_i = lax.fori_loop(0, B, lambda i, a: a + lens_ref[i], 0)
    cnt = jnp.maximum(cnt_i, 1).astype(jnp.float32)
    mean = jnp.sum(sum_ref[...], axis=0) / cnt
    var = jnp.maximum(jnp.sum(sq_ref[...], axis=0) / cnt - mean * mean, 0.0)
    scale_ref[0, :] = w_ref[0, :] * lax.rsqrt(var + EPS)
    shift_ref[0, :] = b_ref[0, :] - mean * scale_ref[0, :]


def _norm_body(lens_ref, x_ref, scale_ref, shift_ref, o_ref):
    b = pl.program_id(0)
    j = pl.program_id(1)
    rel = lens_ref[b] - j * SB
    x = x_ref[0]
    iota = lax.broadcasted_iota(jnp.int32, (SB, 1), 0)
    valid = iota < rel
    o_ref[0] = jnp.where(valid,
                         x * scale_ref[0, :][None, :]
                         + shift_ref[0, :][None, :], 0.0)


@jax.jit
def _vlbn(x, lens32, weight, bias):
    sums, sqs = _sc_stats(x, lens32)
    sums = sums.reshape(NW, D)
    sqs = sqs.reshape(NW, D)

    lens_spec = pl.BlockSpec(memory_space=pltpu.SMEM)
    scale, shift = pl.pallas_call(
        _finalize_body,
        in_specs=[lens_spec] + [pl.BlockSpec((NW, D), lambda: (0, 0))] * 2
                 + [pl.BlockSpec((1, D), lambda: (0, 0))] * 2,
        out_specs=[pl.BlockSpec((1, D), lambda: (0, 0))] * 2,
        out_shape=[jax.ShapeDtypeStruct((1, D), jnp.float32)] * 2,
    )(lens32, sums, sqs, weight.reshape(1, D), bias.reshape(1, D))

    def x_map(b, j, lens):
        lastv = jnp.maximum((lens[b] + SB - 1) // SB - 1, 0)
        return (b, jnp.minimum(j, lastv), 0)

    grid_spec = pltpu.PrefetchScalarGridSpec(
        num_scalar_prefetch=1,
        grid=(B, S_BLKS),
        in_specs=[
            pl.BlockSpec((1, SB, D), x_map),
            pl.BlockSpec((1, D), lambda b, j, lens: (0, 0)),
            pl.BlockSpec((1, D), lambda b, j, lens: (0, 0)),
        ],
        out_specs=pl.BlockSpec((1, SB, D), lambda b, j, lens: (b, j, 0)),
    )
    out = pl.pallas_call(
        _norm_body,
        grid_spec=grid_spec,
        out_shape=jax.ShapeDtypeStruct((B, S, D), jnp.float32),
    )(lens32, x, scale, shift)
    return out


def kernel(inputs, seq_lens, weight, bias):
    lens32 = seq_lens.astype(jnp.int32)
    with jax.enable_x64(False):
        return _vlbn(inputs.astype(jnp.float32), lens32,
                     weight.astype(jnp.float32), bias.astype(jnp.float32))


# speedup vs baseline: 1.3746x; 1.0993x over previous
import functools

import jax
import jax.numpy as jnp
from jax import lax
from jax.experimental import pallas as pl
from jax.experimental.pallas import tpu as pltpu
from jax.experimental.pallas import tpu_sc as plsc

B, S, D = 16, 4096, 512
SB = 4096
S_BLKS = S // SB
EPS = 1e-5

NC, NS, L = 2, 16, 16
NW = NC * NS
CHUNK = 64
NSL = D // L
KG = 4
BSC0 = 10
SB_ST = 2048
ST_BLKS = S // SB_ST


def _sc_stats_body(x_hbm, lens_hbm, sum_hbm, sq_hbm, lens_v, buf, acc_s,
                   acc_q, sem):
    wid = lax.axis_index("s") * NC + lax.axis_index("c")
    pltpu.sync_copy(lens_hbm, lens_v)
    lv = lens_v[...]
    total = jnp.int32(0)
    for b in range(BSC0, B):
        total = total + lv[b]
    tstart = wid * total // NW
    tend = (wid + 1) * total // NW

    for k in range(NSL):
        acc_s[pl.ds(k * L, L)] = jnp.zeros((L,), jnp.float32)
        acc_q[pl.ds(k * L, L)] = jnp.zeros((L,), jnp.float32)

    def row_body(b, pre_b):
        len_b = jnp.int32(0)
        for i in range(BSC0, B):
            len_b = len_b + jnp.where(b == i, lv[i], 0)
        lo = jnp.maximum(tstart - pre_b, 0)
        hi = jnp.minimum(tend - pre_b, len_b)

        s00 = jnp.minimum((lo // 8) * 8, S - CHUNK)
        nch = jnp.where(hi > lo, (hi - s00 + CHUNK - 1) // CHUNK, 0)

        def s0_of(i):
            return pl.multiple_of(jnp.minimum(s00 + i * CHUNK, S - CHUNK), 8)

        def start_dma(i):
            slot = lax.rem(i, 2)
            pltpu.make_async_copy(x_hbm.at[b, pl.ds(s0_of(i), CHUNK)],
                                  buf.at[slot], sem.at[slot]).start()

        @pl.when(nch > 0)
        def _prime():
            start_dma(0)

        def chunk_body(ci, _):
            slot = lax.rem(ci, 2)
            pltpu.make_async_copy(x_hbm.at[b, pl.ds(s0_of(ci), CHUNK)],
                                  buf.at[slot], sem.at[slot]).wait()

            @pl.when(ci + 1 < nch)
            def _prefetch():
                start_dma(ci + 1)

            s0 = s0_of(ci)
            p = jnp.where(ci == 0, lo, s00 + ci * CHUNK)
            off = p - s0
            lim = jnp.minimum(hi - s0, CHUNK)

            def zero_tok(t, _):
                for k in range(NSL):
                    buf[slot, t, pl.ds(k * L, L)] = jnp.zeros((L,),
                                                              jnp.float32)
                return 0

            lax.fori_loop(0, off, zero_tok, 0)
            lax.fori_loop(lim, CHUNK, zero_tok, 0)

            for g in range(NSL // KG):
                sls = [pl.ds((g * KG + u) * L, L) for u in range(KG)]

                def tok(t, carry, sls=sls, slot=slot):
                    vs = [buf[slot, t, sl] for sl in sls]
                    ss = [carry[u] + vs[u] for u in range(KG)]
                    qs = [carry[KG + u] + vs[u] * vs[u] for u in range(KG)]
                    return tuple(ss + qs)

                init = tuple([acc_s[sl] for sl in sls]
                             + [acc_q[sl] for sl in sls])
                res = lax.fori_loop(0, CHUNK, tok, init, unroll=2)
                for u in range(KG):
                    acc_s[sls[u]] = res[u]
                    acc_q[sls[u]] = res[KG + u]
            return 0

        lax.fori_loop(0, nch, chunk_body, 0)
        return pre_b + len_b

    lax.fori_loop(BSC0, B, row_body, jnp.int32(0))
    wbase = pl.multiple_of(wid * D, 8)
    pltpu.sync_copy(acc_s, sum_hbm.at[pl.ds(wbase, D)])
    pltpu.sync_copy(acc_q, sq_hbm.at[pl.ds(wbase, D)])


_sc_stats = functools.partial(
    pl.kernel,
    out_type=[
        jax.ShapeDtypeStruct((NW * D,), jnp.float32),
        jax.ShapeDtypeStruct((NW * D,), jnp.float32),
    ],
    mesh=plsc.VectorSubcoreMesh(core_axis_name="c", subcore_axis_name="s",
                                num_cores=NC, num_subcores=NS),
    scratch_types=[
        pltpu.VMEM((L,), jnp.int32),
        pltpu.VMEM((2, CHUNK, D), jnp.float32),
        pltpu.VMEM((D,), jnp.float32),
        pltpu.VMEM((D,), jnp.float32),
        pltpu.SemaphoreType.DMA((2,)),
    ],
)(_sc_stats_body)


def _tc_stats_body(lens_ref, x_ref, sum_ref, sq_ref):
    b = pl.program_id(0)
    j = pl.program_id(1)

    @pl.when(jnp.logical_and(b == 0, j == 0))
    def _init():
        sum_ref[...] = jnp.zeros_like(sum_ref)
        sq_ref[...] = jnp.zeros_like(sq_ref)

    rel = lens_ref[b] - j * SB_ST
    x = x_ref[0]
    iota = lax.broadcasted_iota(jnp.int32, (SB_ST, 1), 0)
    xm = jnp.where(iota < rel, x, 0.0)
    sum_ref[0, :] += xm.sum(axis=0)
    sq_ref[0, :] += (xm * xm).sum(axis=0)


def _finalize_body(lens_ref, tsum_ref, tsq_ref, sum_ref, sq_ref, w_ref,
                   b_ref, scale_ref, shift_ref):
    cnt_i = lax.fori_loop(0, B, lambda i, a: a + lens_ref[i], 0)
    cnt = jnp.maximum(cnt_i, 1).astype(jnp.float32)
    mean = (tsum_ref[0, :] + jnp.sum(sum_ref[...], axis=0)) / cnt
    var = jnp.maximum(
        (tsq_ref[0, :] + jnp.sum(sq_ref[...], axis=0)) / cnt - mean * mean,
        0.0)
    scale_ref[0, :] = w_ref[0, :] * lax.rsqrt(var + EPS)
    shift_ref[0, :] = b_ref[0, :] - mean * scale_ref[0, :]


def _norm_body(lens_ref, x_ref, scale_ref, shift_ref, o_ref):
    b = pl.program_id(0)
    j = pl.program_id(1)
    rel = lens_ref[b] - j * SB
    x = x_ref[0]
    iota = lax.broadcasted_iota(jnp.int32, (SB, 1), 0)
    valid = iota < rel
    o_ref[0] = jnp.where(valid,
                         x * scale_ref[0, :][None, :]
                         + shift_ref[0, :][None, :], 0.0)


@jax.jit
def _vlbn(x, lens32, weight, bias):
    sums, sqs = _sc_stats(x, lens32)
    sums = sums.reshape(NW, D)
    sqs = sqs.reshape(NW, D)

    def xst_map(b, j, lens):
        lastv = jnp.maximum((lens[b] + SB_ST - 1) // SB_ST - 1, 0)
        return (b, jnp.minimum(j, lastv), 0)

    st_grid_spec = pltpu.PrefetchScalarGridSpec(
        num_scalar_prefetch=1,
        grid=(BSC0, ST_BLKS),
        in_specs=[pl.BlockSpec((1, SB_ST, D), xst_map)],
        out_specs=[pl.BlockSpec((1, D), lambda b, j, lens: (0, 0))] * 2,
    )
    tsum, tsq = pl.pallas_call(
        _tc_stats_body,
        grid_spec=st_grid_spec,
        out_shape=[jax.ShapeDtypeStruct((1, D), jnp.float32)] * 2,
    )(lens32, x)

    lens_spec = pl.BlockSpec(memory_space=pltpu.SMEM)
    scale, shift = pl.pallas_call(
        _finalize_body,
        in_specs=[lens_spec] + [pl.BlockSpec((1, D), lambda: (0, 0))] * 2
                 + [pl.BlockSpec((NW, D), lambda: (0, 0))] * 2
                 + [pl.BlockSpec((1, D), lambda: (0, 0))] * 2,
        out_specs=[pl.BlockSpec((1, D), lambda: (0, 0))] * 2,
        out_shape=[jax.ShapeDtypeStruct((1, D), jnp.float32)] * 2,
    )(lens32, tsum, tsq, sums, sqs, weight.reshape(1, D), bias.reshape(1, D))

    def x_map(b, j, lens):
        lastv = jnp.maximum((lens[b] + SB - 1) // SB - 1, 0)
        return (b, jnp.minimum(j, lastv), 0)

    grid_spec = pltpu.PrefetchScalarGridSpec(
        num_scalar_prefetch=1,
        grid=(B, S_BLKS),
        in_specs=[
            pl.BlockSpec((1, SB, D), x_map),
            pl.BlockSpec((1, D), lambda b, j, lens: (0, 0)),
            pl.BlockSpec((1, D), lambda b, j, lens: (0, 0)),
        ],
        out_specs=pl.BlockSpec((1, SB, D), lambda b, j, lens: (b, j, 0)),
    )
    out = pl.pallas_call(
        _norm_body,
        grid_spec=grid_spec,
        out_shape=jax.ShapeDtypeStruct((B, S, D), jnp.float32),
    )(lens32, x, scale, shift)
    return out


def kernel(inputs, seq_lens, weight, bias):
    lens32 = seq_lens.astype(jnp.int32)
    with jax.enable_x64(False):
        return _vlbn(inputs.astype(jnp.float32), lens32,
                     weight.astype(jnp.float32), bias.astype(jnp.float32))
